# Initial kernel scaffold; baseline (speedup 1.0000x reference)
import functools

import jax
import jax.numpy as jnp
from jax import lax
from jax.experimental import pallas as pl
from jax.experimental.pallas import tpu as pltpu
from jax.experimental.pallas import tpu_sc as plsc

B = 16384
L = 50
NINP = 64
TOT = B * L
NW = 32
CHUNK = 128
PER_W = TOT // NW
NCH = PER_W // CHUNK

_mesh = plsc.VectorSubcoreMesh(core_axis_name="c", subcore_axis_name="s")


@functools.partial(
    pl.kernel,
    mesh=_mesh,
    out_type=jax.ShapeDtypeStruct((TOT, NINP), jnp.float32),
    scratch_types=[
        pltpu.VMEM((NCH, CHUNK), jnp.int32),
        pltpu.VMEM((2, CHUNK, NINP), jnp.float32),
        pltpu.SemaphoreType.DMA,
        pltpu.SemaphoreType.DMA,
    ],
)
def _emb_lookup(idx_hbm, table_hbm, out_hbm, idx_v, rows_v, gsem, ssem):
    wid = lax.axis_index("s") * 2 + lax.axis_index("c")
    chunk0 = wid * NCH
    pltpu.sync_copy(idx_hbm.at[pl.ds(chunk0, NCH)], idx_v)

    pltpu.async_copy(table_hbm.at[idx_v.at[0]], rows_v.at[0], gsem)

    def step(g, _):
        slot = lax.rem(g, 2)
        nxt = 1 - slot
        pltpu.make_async_copy(
            table_hbm.at[idx_v.at[g]], rows_v.at[slot], gsem
        ).wait()

        @pl.when(g + 1 < NCH)
        def _():
            pltpu.async_copy(
                table_hbm.at[idx_v.at[g + 1]], rows_v.at[nxt], gsem
            )

        @pl.when(g >= 1)
        def _():
            pltpu.make_async_copy(
                rows_v.at[nxt],
                out_hbm.at[pl.ds((chunk0 + g - 1) * CHUNK, CHUNK)],
                ssem,
            ).wait()

        pltpu.async_copy(
            rows_v.at[slot],
            out_hbm.at[pl.ds((chunk0 + g) * CHUNK, CHUNK)],
            ssem,
        )
        return 0

    lax.fori_loop(0, NCH, step, 0)
    pltpu.make_async_copy(
        rows_v.at[(NCH - 1) % 2],
        out_hbm.at[pl.ds((chunk0 + NCH - 1) * CHUNK, CHUNK)],
        ssem,
    ).wait()


def kernel(input, weight):
    idx = input.reshape(TOT // CHUNK, CHUNK)
    out = _emb_lookup(idx, weight)
    return out.reshape(B, L, NINP)


# speedup vs baseline: 1.7488x; 1.7488x over previous
import functools

import jax
import jax.numpy as jnp
from jax import lax
from jax.experimental import pallas as pl
from jax.experimental.pallas import tpu as pltpu
from jax.experimental.pallas import tpu_sc as plsc

B = 16384
L = 50
NINP = 64
TOT = B * L
NW = 32
CHUNK = 128
PER_W = TOT // NW
NCH = PER_W // CHUNK
NH = NCH // 2

_mesh = plsc.VectorSubcoreMesh(core_axis_name="c", subcore_axis_name="s")


@functools.partial(
    pl.kernel,
    mesh=_mesh,
    compiler_params=pltpu.CompilerParams(use_tc_tiling_on_sc=False),
    out_type=jax.ShapeDtypeStruct((TOT, NINP), jnp.float32),
    scratch_types=[
        pltpu.VMEM((NCH, CHUNK), jnp.int32),
        pltpu.VMEM((CHUNK, NINP), jnp.float32),
        pltpu.VMEM((CHUNK, NINP), jnp.float32),
        pltpu.SemaphoreType.DMA,
        pltpu.SemaphoreType.DMA,
    ],
)
def _emb_lookup(idx_hbm, table_hbm, out_hbm, idx_v, buf0, buf1, gsem, ssem):
    wid = lax.axis_index("s") * 2 + lax.axis_index("c")
    chunk0 = wid * NCH
    pltpu.sync_copy(idx_hbm.at[pl.ds(chunk0, NCH)], idx_v)

    pltpu.async_copy(table_hbm.at[idx_v.at[0]], buf0, gsem)

    def step(h, _):
        c0 = 2 * h
        c1 = 2 * h + 1

        pltpu.make_async_copy(table_hbm.at[idx_v.at[c0]], buf0, gsem).wait()

        @pl.when(h >= 1)
        def _():
            pltpu.make_async_copy(
                buf1, out_hbm.at[pl.ds((chunk0 + 2 * h - 1) * CHUNK, CHUNK)],
                ssem,
            ).wait()

        pltpu.async_copy(table_hbm.at[idx_v.at[c1]], buf1, gsem)
        pltpu.async_copy(
            buf0, out_hbm.at[pl.ds((chunk0 + c0) * CHUNK, CHUNK)], ssem
        )

        pltpu.make_async_copy(table_hbm.at[idx_v.at[c1]], buf1, gsem).wait()
        pltpu.make_async_copy(
            buf0, out_hbm.at[pl.ds((chunk0 + c0) * CHUNK, CHUNK)], ssem
        ).wait()

        @pl.when(h + 1 < NH)
        def _():
            pltpu.async_copy(table_hbm.at[idx_v.at[c0 + 2]], buf0, gsem)

        pltpu.async_copy(
            buf1, out_hbm.at[pl.ds((chunk0 + c1) * CHUNK, CHUNK)], ssem
        )
        return 0

    lax.fori_loop(0, NH, step, 0)
    pltpu.make_async_copy(
        buf1, out_hbm.at[pl.ds((chunk0 + NCH - 1) * CHUNK, CHUNK)], ssem
    ).wait()


def kernel(input, weight):
    idx = input.reshape(TOT // CHUNK, CHUNK)
    out = _emb_lookup(idx, weight)
    return out.reshape(B, L, NINP)


# speedup vs baseline: 1.8757x; 1.0725x over previous
import functools

import jax
import jax.numpy as jnp
from jax import lax
from jax.experimental import pallas as pl
from jax.experimental.pallas import tpu as pltpu
from jax.experimental.pallas import tpu_sc as plsc

B = 16384
L = 50
NINP = 64
TOT = B * L
NW = 32
CHUNK = 128
PER_W = TOT // NW
NCH = PER_W // CHUNK
NH = NCH // 2

_mesh = plsc.VectorSubcoreMesh(core_axis_name="c", subcore_axis_name="s")


NBUF = 8
DEPTH = 4


@functools.partial(
    pl.kernel,
    mesh=_mesh,
    compiler_params=pltpu.CompilerParams(use_tc_tiling_on_sc=False),
    out_type=jax.ShapeDtypeStruct((TOT, NINP), jnp.float32),
    scratch_types=[
        pltpu.VMEM((NCH, CHUNK), jnp.int32),
        pltpu.VMEM((NBUF, CHUNK, NINP), jnp.float32),
        pltpu.SemaphoreType.DMA,
        pltpu.SemaphoreType.DMA,
    ],
)
def _emb_lookup(idx_hbm, table_hbm, out_hbm, idx_v, bufs, gsem, ssem):
    wid = lax.axis_index("s") * 2 + lax.axis_index("c")
    chunk0 = wid * NCH
    pltpu.sync_copy(idx_hbm.at[pl.ds(chunk0, NCH)], idx_v)

    for j in range(DEPTH):
        pltpu.async_copy(table_hbm.at[idx_v.at[j]], bufs.at[j], gsem)

    NG = NCH // DEPTH

    def block(h, _):
        for p in range(2):
            g = 2 * h + p
            me = p * DEPTH
            other = (1 - p) * DEPTH

            @pl.when(g >= 1)
            def _():
                for j in range(DEPTH):
                    pltpu.make_async_copy(
                        bufs.at[other + j],
                        out_hbm.at[
                            pl.ds((chunk0 + (g - 1) * DEPTH + j) * CHUNK,
                                  CHUNK)
                        ],
                        ssem,
                    ).wait()

            @pl.when(g + 1 < NG)
            def _():
                for j in range(DEPTH):
                    pltpu.async_copy(
                        table_hbm.at[idx_v.at[(g + 1) * DEPTH + j]],
                        bufs.at[other + j],
                        gsem,
                    )

            for j in range(DEPTH):
                pltpu.make_async_copy(
                    table_hbm.at[idx_v.at[g * DEPTH + j]],
                    bufs.at[me + j],
                    gsem,
                ).wait()
            for j in range(DEPTH):
                pltpu.async_copy(
                    bufs.at[me + j],
                    out_hbm.at[
                        pl.ds((chunk0 + g * DEPTH + j) * CHUNK, CHUNK)
                    ],
                    ssem,
                )
        return 0

    lax.fori_loop(0, NG // 2, block, 0)
    for j in range(DEPTH):
        pltpu.make_async_copy(
            bufs.at[DEPTH + j],
            out_hbm.at[pl.ds((chunk0 + (NG - 1) * DEPTH + j) * CHUNK, CHUNK)],
            ssem,
        ).wait()


def kernel(input, weight):
    idx = input.reshape(TOT // CHUNK, CHUNK)
    out = _emb_lookup(idx, weight)
    return out.reshape(B, L, NINP)


# speedup vs baseline: 1.8790x; 1.0018x over previous
import functools

import jax
import jax.numpy as jnp
from jax import lax
from jax.experimental import pallas as pl
from jax.experimental.pallas import tpu as pltpu
from jax.experimental.pallas import tpu_sc as plsc

B = 16384
L = 50
NINP = 64
TOT = B * L
NW = 32
PER_W = TOT // NW
SLAB = 320
NSLAB = PER_W // SLAB
NBUF = 4

_mesh = plsc.VectorSubcoreMesh(core_axis_name="c", subcore_axis_name="s")


@functools.partial(
    pl.kernel,
    mesh=_mesh,
    compiler_params=pltpu.CompilerParams(use_tc_tiling_on_sc=False),
    out_type=jax.ShapeDtypeStruct((TOT, NINP), jnp.float32),
    scratch_types=[
        pltpu.VMEM((PER_W,), jnp.int32),
        pltpu.VMEM((NBUF, SLAB, NINP), jnp.float32),
        pltpu.SemaphoreType.DMA,
        pltpu.SemaphoreType.DMA,
    ],
)
def _emb_lookup(idx_hbm, table_hbm, out_hbm, idx_v, bufs, gsem, ssem):
    wid = lax.axis_index("s") * 2 + lax.axis_index("c")
    base = wid * PER_W
    pltpu.sync_copy(idx_hbm.at[pl.ds(base, PER_W)], idx_v)

    def gather(s, b):
        pltpu.async_copy(
            table_hbm.at[idx_v.at[pl.ds(s * SLAB, SLAB)]], bufs.at[b], gsem
        )

    def scatter_copy(s, b):
        return pltpu.make_async_copy(
            bufs.at[b], out_hbm.at[pl.ds(base + s * SLAB, SLAB)], ssem
        )

    gather(0, 0)
    gather(1, 1)

    def block(h, _):
        for j in range(NBUF):
            g = h * NBUF + j
            pltpu.make_async_copy(
                table_hbm.at[idx_v.at[pl.ds(g * SLAB, SLAB)]],
                bufs.at[j],
                gsem,
            ).wait()
            scatter_copy(g, j).start()

            @pl.when(g >= 2)
            def _():
                scatter_copy(g - 2, (j + 2) % NBUF).wait()

            @pl.when(g + 2 < NSLAB)
            def _():
                gather(g + 2, (j + 2) % NBUF)
        return 0

    lax.fori_loop(0, NSLAB // NBUF, block, 0)
    scatter_copy(NSLAB - 2, (NSLAB - 2) % NBUF).wait()
    scatter_copy(NSLAB - 1, (NSLAB - 1) % NBUF).wait()


def kernel(input, weight):
    idx = input.reshape(TOT)
    out = _emb_lookup(idx, weight)
    return out.reshape(B, L, NINP)
